# baseline (speedup 1.0000x reference)
import jax
import jax.numpy as jnp
from jax.experimental import pallas as pl

N_CODE = 1024
DIM = 64
TOK = 1024
NB = 16


def _vq_body(z_ref, w_ref, zq_ref, idx_ref, loss_ref):
    b = pl.program_id(0)
    zc = z_ref[0].reshape(DIM, TOK)
    w = w_ref[...]
    zt = zc.T
    zn = jnp.sum(zt * zt, axis=1, keepdims=True)
    wn = jnp.sum(w * w, axis=1)
    mm = jax.lax.dot_general(zt, w, (((1,), (1,)), ((), ())),
                             preferred_element_type=jnp.float32)
    dist = (zn + wn) - 2.0 * mm
    m = jnp.min(dist, axis=1, keepdims=True)
    iota_j = jax.lax.broadcasted_iota(jnp.int32, dist.shape, 1)
    idx = jnp.min(jnp.where(dist == m, iota_j, N_CODE), axis=1)
    idx_ref[0] = idx.reshape(32, 32)
    e = (jax.lax.broadcasted_iota(jnp.int32, (N_CODE, TOK), 0)
         == idx[None, :]).astype(jnp.bfloat16)
    zq_t = jax.lax.dot_general(w.astype(jnp.bfloat16), e,
                               (((0,), (0,)), ((), ())),
                               preferred_element_type=jnp.float32)
    zq_ref[0] = zq_t.reshape(DIM, 32, 32)
    part = jnp.sum(m, axis=(0, 1), keepdims=True)

    @pl.when(b == 0)
    def _init():
        loss_ref[...] = jnp.zeros((1, 1), jnp.float32)

    loss_ref[...] += part

    @pl.when(b == NB - 1)
    def _fin():
        loss_ref[...] = loss_ref[...] / (NB * TOK * DIM)


def kernel(z, W):
    B, C, H, Wd = z.shape
    zq, idx, loss = pl.pallas_call(
        _vq_body,
        grid=(B,),
        in_specs=[
            pl.BlockSpec((1, C, H, Wd), lambda b: (b, 0, 0, 0)),
            pl.BlockSpec((N_CODE, DIM), lambda b: (0, 0)),
        ],
        out_specs=[
            pl.BlockSpec((1, C, H, Wd), lambda b: (b, 0, 0, 0)),
            pl.BlockSpec((1, H, Wd), lambda b: (b, 0, 0)),
            pl.BlockSpec((1, 1), lambda b: (0, 0)),
        ],
        out_shape=[
            jax.ShapeDtypeStruct((B, C, H, Wd), jnp.float32),
            jax.ShapeDtypeStruct((B, H, Wd), jnp.int32),
            jax.ShapeDtypeStruct((1, 1), jnp.float32),
        ],
    )(z, W)
    codebook_loss = loss.reshape(())
    commitment_loss = 0.25 * codebook_loss
    return (zq, codebook_loss, commitment_loss, idx)


# speedup vs baseline: 1.4333x; 1.4333x over previous
import jax
import jax.numpy as jnp
from jax.experimental import pallas as pl

N_CODE = 1024
DIM = 64
TOK = 1024
NB = 16
GB = 2
T = GB * TOK


def _vq_body(z_ref, w_ref, zq_ref, idx_ref, loss_ref):
    b = pl.program_id(0)
    w = w_ref[...]
    zt = jnp.concatenate([z_ref[k].T for k in range(GB)], axis=0)
    zn = jnp.sum(zt * zt, axis=1, keepdims=True)
    wn = jnp.sum(w * w, axis=1)
    mm = jax.lax.dot_general(zt, w, (((1,), (1,)), ((), ())),
                             preferred_element_type=jnp.float32)
    dist = (zn + wn) - 2.0 * mm
    m = jnp.min(dist, axis=1, keepdims=True)
    iota_j = jax.lax.broadcasted_iota(jnp.int32, dist.shape, 1)
    idx = jnp.min(jnp.where(dist == m, iota_j, N_CODE), axis=1)
    e = (jax.lax.broadcasted_iota(jnp.int32, (N_CODE, T), 0)
         == idx[None, :]).astype(jnp.bfloat16)
    zq_t = jax.lax.dot_general(w.astype(jnp.bfloat16), e,
                               (((0,), (0,)), ((), ())),
                               preferred_element_type=jnp.float32)
    for k in range(GB):
        idx_ref[k, 0, :] = idx[k * TOK:(k + 1) * TOK]
        zq_ref[k] = zq_t[:, k * TOK:(k + 1) * TOK]
    part = jnp.sum(m, axis=(0, 1), keepdims=True)

    @pl.when(b == 0)
    def _init():
        loss_ref[...] = jnp.zeros((1, 1), jnp.float32)

    loss_ref[...] += part

    @pl.when(b == NB // GB - 1)
    def _fin():
        loss_ref[...] = loss_ref[...] / (NB * TOK * DIM)


def kernel(z, W):
    B, C, H, Wd = z.shape
    z3 = z.reshape(B, C, H * Wd)
    zq3, idx3, loss = pl.pallas_call(
        _vq_body,
        grid=(B // GB,),
        in_specs=[
            pl.BlockSpec((GB, C, H * Wd), lambda b: (b, 0, 0)),
            pl.BlockSpec((N_CODE, DIM), lambda b: (0, 0)),
        ],
        out_specs=[
            pl.BlockSpec((GB, C, H * Wd), lambda b: (b, 0, 0)),
            pl.BlockSpec((GB, 1, H * Wd), lambda b: (b, 0, 0)),
            pl.BlockSpec((1, 1), lambda b: (0, 0)),
        ],
        out_shape=[
            jax.ShapeDtypeStruct((B, C, H * Wd), jnp.float32),
            jax.ShapeDtypeStruct((B, 1, H * Wd), jnp.int32),
            jax.ShapeDtypeStruct((1, 1), jnp.float32),
        ],
    )(z3, W)
    z_q = zq3.reshape(B, C, H, Wd)
    codebook_loss = loss.reshape(())
    commitment_loss = 0.25 * codebook_loss
    min_encoding_indices = idx3.reshape(B, H, Wd)
    return (z_q, codebook_loss, commitment_loss, min_encoding_indices)


# speedup vs baseline: 1.5360x; 1.0717x over previous
import jax
import jax.numpy as jnp
from jax.experimental import pallas as pl

N_CODE = 1024
DIM = 64
TOK = 1024
NB = 16
GB = 4
T = GB * TOK


def _vq_body(z_ref, w_ref, zq_ref, idx_ref, loss_ref):
    b = pl.program_id(0)
    w = w_ref[...]
    zt = jnp.concatenate([z_ref[k].T for k in range(GB)], axis=0)
    zn = jnp.sum(zt * zt, axis=1, keepdims=True)
    wn = jnp.sum(w * w, axis=1)
    mm2 = jax.lax.dot_general(zt + zt, w, (((1,), (1,)), ((), ())),
                              preferred_element_type=jnp.float32)
    dist = (zn + wn) - mm2
    m = jnp.min(dist, axis=1, keepdims=True)
    iota_j = jax.lax.broadcasted_iota(jnp.int32, dist.shape, 1)
    idx = jnp.min(jnp.where(dist == m, iota_j, N_CODE), axis=1)
    e = (jax.lax.broadcasted_iota(jnp.int32, (N_CODE, T), 0)
         == idx[None, :]).astype(jnp.bfloat16)
    zq_t = jax.lax.dot_general(w.astype(jnp.bfloat16), e,
                               (((0,), (0,)), ((), ())),
                               preferred_element_type=jnp.float32)
    for k in range(GB):
        idx_ref[k, 0, :] = idx[k * TOK:(k + 1) * TOK]
        zq_ref[k] = zq_t[:, k * TOK:(k + 1) * TOK]
    part = jnp.sum(m, axis=(0, 1), keepdims=True)

    @pl.when(b == 0)
    def _init():
        loss_ref[...] = jnp.zeros((1, 1), jnp.float32)

    loss_ref[...] += part

    @pl.when(b == NB // GB - 1)
    def _fin():
        loss_ref[...] = loss_ref[...] / (NB * TOK * DIM)


def kernel(z, W):
    B, C, H, Wd = z.shape
    z3 = z.reshape(B, C, H * Wd)
    zq3, idx3, loss = pl.pallas_call(
        _vq_body,
        grid=(B // GB,),
        in_specs=[
            pl.BlockSpec((GB, C, H * Wd), lambda b: (b, 0, 0)),
            pl.BlockSpec((N_CODE, DIM), lambda b: (0, 0)),
        ],
        out_specs=[
            pl.BlockSpec((GB, C, H * Wd), lambda b: (b, 0, 0)),
            pl.BlockSpec((GB, 1, H * Wd), lambda b: (b, 0, 0)),
            pl.BlockSpec((1, 1), lambda b: (0, 0)),
        ],
        out_shape=[
            jax.ShapeDtypeStruct((B, C, H * Wd), jnp.float32),
            jax.ShapeDtypeStruct((B, 1, H * Wd), jnp.int32),
            jax.ShapeDtypeStruct((1, 1), jnp.float32),
        ],
    )(z3, W)
    z_q = zq3.reshape(B, C, H, Wd)
    codebook_loss = loss.reshape(())
    commitment_loss = 0.25 * codebook_loss
    min_encoding_indices = idx3.reshape(B, H, Wd)
    return (z_q, codebook_loss, commitment_loss, min_encoding_indices)
